# Initial kernel scaffold; baseline (speedup 1.0000x reference)
import jax
import jax.numpy as jnp
from jax.experimental import pallas as pl


def kernel(x, embed, Wih_f, Whh_f, bih_f, bhh_f, Wih_b, Whh_b, bih_b, bhh_b, fc_W, fc_b):
    raise NotImplementedError("write your pallas kernel here")


# speedup vs baseline: 3.3682x; 3.3682x over previous
import functools

import jax
import jax.numpy as jnp
from jax import lax
from jax.experimental import pallas as pl
from jax.experimental.pallas import tpu as pltpu
from jax.experimental.pallas import tpu_sc as plsc

B = 1024
L = 200
E = 64
EP = 128
H = 80
HP = 128
G4 = 4 * HP
NCLS = 15
F32 = jnp.float32


@functools.lru_cache(maxsize=None)
def _make_gather():
  info = plsc.get_sparse_core_info()
  nw = info.num_cores * info.num_subcores
  tot = B * L
  per_w = tot // nw
  ch = 800
  n_ch = per_w // ch

  @functools.partial(
      pl.kernel,
      out_type=jax.ShapeDtypeStruct((tot, EP), F32),
      mesh=plsc.VectorSubcoreMesh(core_axis_name="c", subcore_axis_name="s"),
      scratch_types=[
          pltpu.VMEM((ch,), jnp.int32),
          pltpu.VMEM((ch, EP), F32),
          pltpu.SemaphoreType.DMA,
      ],
  )
  def gather_k(table_hbm, idx_hbm, out_hbm, idx_v, buf, sem):
    wid = lax.axis_index("s") * info.num_cores + lax.axis_index("c")
    base = wid * per_w
    for j in range(n_ch):
      off = base + j * ch
      pltpu.sync_copy(idx_hbm.at[pl.ds(off, ch)], idx_v)
      pltpu.async_copy(table_hbm.at[idx_v], buf, sem).wait()
      pltpu.sync_copy(buf, out_hbm.at[pl.ds(off, ch)])

  return gather_k


def _lstm_step(x_t, h_prev, c_prev, wih_ref, whh_ref, b_ref):
  g = jnp.dot(x_t, wih_ref[...], preferred_element_type=F32)
  g = g + jnp.dot(h_prev, whh_ref[...], preferred_element_type=F32)
  g = g + b_ref[...]
  ig = jax.nn.sigmoid(g[:, 0:HP])
  fg = jax.nn.sigmoid(g[:, HP:2 * HP])
  gg = jnp.tanh(g[:, 2 * HP:3 * HP])
  og = jax.nn.sigmoid(g[:, 3 * HP:4 * HP])
  c_new = fg * c_prev + ig * gg
  h_new = og * jnp.tanh(c_new)
  return h_new, c_new


def _bwd_body(e_ref, wih_ref, whh_ref, b_ref, out_ref, h_s, c_s):
  i = pl.program_id(0)

  @pl.when(i == 0)
  def _():
    h_s[...] = jnp.zeros_like(h_s)
    c_s[...] = jnp.zeros_like(c_s)

  h_new, c_new = _lstm_step(e_ref[0], h_s[...], c_s[...],
                            wih_ref, whh_ref, b_ref)
  h_s[...] = h_new
  c_s[...] = c_new
  out_ref[0] = h_new[:, :H]


def _fwd_body(e_ref, hb_ref, wih_ref, whh_ref, b_ref, fcwf_ref, fcwb_ref,
              fcb_ref, out_ref, h_s, c_s, mxf, smf, mxb, smb):
  i = pl.program_id(0)

  @pl.when(i == 0)
  def _():
    h_s[...] = jnp.zeros_like(h_s)
    c_s[...] = jnp.zeros_like(c_s)
    mxf[...] = jnp.full_like(mxf, -jnp.inf)
    smf[...] = jnp.zeros_like(smf)
    mxb[...] = jnp.full_like(mxb, -jnp.inf)
    smb[...] = jnp.zeros_like(smb)

  h_new, c_new = _lstm_step(e_ref[0], h_s[...], c_s[...],
                            wih_ref, whh_ref, b_ref)
  h_s[...] = h_new
  c_s[...] = c_new

  hb = hb_ref[0]
  s1 = jnp.sum(h_new, axis=1, keepdims=True) + jnp.sum(hb, axis=1, keepdims=True)
  s2 = (jnp.sum(h_new * h_new, axis=1, keepdims=True)
        + jnp.sum(hb * hb, axis=1, keepdims=True))
  mu = s1 * (1.0 / (2 * H))
  var = s2 * (1.0 / (2 * H)) - mu * mu
  rstd = lax.rsqrt(var + 1e-5)
  lnf = (h_new - mu) * rstd
  lnb = (hb - mu) * rstd
  mxf[...] = jnp.maximum(mxf[...], lnf)
  smf[...] = smf[...] + lnf
  mxb[...] = jnp.maximum(mxb[...], lnb)
  smb[...] = smb[...] + lnb

  @pl.when(i == L - 1)
  def _():
    zf = 0.5 * mxf[...] + (0.5 / L) * smf[...]
    zb = 0.5 * mxb[...] + (0.5 / L) * smb[...]
    logits = (jnp.dot(zf, fcwf_ref[...], preferred_element_type=F32)
              + jnp.dot(zb, fcwb_ref[...], preferred_element_type=F32)
              + fcb_ref[...])
    m = jnp.max(logits, axis=1, keepdims=True)
    lse = jnp.log(jnp.sum(jnp.exp(logits - m), axis=1, keepdims=True)) + m
    out_ref[...] = logits - lse


def _prep_gates(Wih, Whh, bih, bhh):
  wihT = Wih.T
  whhT = Whh.T
  bb = bih + bhh
  wih = jnp.zeros((EP, G4), F32)
  whh = jnp.zeros((HP, G4), F32)
  b = jnp.zeros((1, G4), F32)
  for g in range(4):
    wih = wih.at[:E, g * HP:g * HP + H].set(wihT[:, g * H:(g + 1) * H])
    whh = whh.at[:H, g * HP:g * HP + H].set(whhT[:, g * H:(g + 1) * H])
    b = b.at[0, g * HP:g * HP + H].set(bb[g * H:(g + 1) * H])
  return wih, whh, b


_FULL = lambda shape: pl.BlockSpec(shape, lambda i: tuple(0 for _ in shape))


def kernel(x, embed, Wih_f, Whh_f, bih_f, bhh_f, Wih_b, Whh_b, bih_b, bhh_b,
           fc_W, fc_b):
  idx = x.T.reshape(-1).astype(jnp.int32)
  embed_p = jnp.pad(embed.astype(F32), ((0, 0), (0, EP - E)))
  e = _make_gather()(embed_p, idx).reshape(L, B, EP)

  wih_b_, whh_b_, b_b_ = _prep_gates(Wih_b, Whh_b, bih_b, bhh_b)
  wih_f_, whh_f_, b_f_ = _prep_gates(Wih_f, Whh_f, bih_f, bhh_f)

  h_b = pl.pallas_call(
      _bwd_body,
      grid=(L,),
      in_specs=[
          pl.BlockSpec((1, B, EP), lambda i: (L - 1 - i, 0, 0)),
          _FULL((EP, G4)),
          _FULL((HP, G4)),
          _FULL((1, G4)),
      ],
      out_specs=pl.BlockSpec((1, B, H), lambda i: (L - 1 - i, 0, 0)),
      out_shape=jax.ShapeDtypeStruct((L, B, H), F32),
      scratch_shapes=[pltpu.VMEM((B, HP), F32), pltpu.VMEM((B, HP), F32)],
  )(e, wih_b_, whh_b_, b_b_)

  fcwf = jnp.zeros((HP, NCLS), F32).at[:H, :].set(fc_W[:, :H].T)
  fcwb = fc_W[:, H:].T
  fcb = fc_b.reshape(1, NCLS)

  out = pl.pallas_call(
      _fwd_body,
      grid=(L,),
      in_specs=[
          pl.BlockSpec((1, B, EP), lambda i: (i, 0, 0)),
          pl.BlockSpec((1, B, H), lambda i: (i, 0, 0)),
          _FULL((EP, G4)),
          _FULL((HP, G4)),
          _FULL((1, G4)),
          _FULL((HP, NCLS)),
          _FULL((H, NCLS)),
          _FULL((1, NCLS)),
      ],
      out_specs=pl.BlockSpec((B, NCLS), lambda i: (0, 0)),
      out_shape=jax.ShapeDtypeStruct((B, NCLS), F32),
      scratch_shapes=[pltpu.VMEM((B, HP), F32), pltpu.VMEM((B, HP), F32),
                      pltpu.VMEM((B, HP), F32), pltpu.VMEM((B, HP), F32),
                      pltpu.VMEM((B, H), F32), pltpu.VMEM((B, H), F32)],
  )(e, h_b, wih_f_, whh_f_, b_f_, fcwf, fcwb, fcb)

  return out
